# Initial kernel scaffold; baseline (speedup 1.0000x reference)
import jax
import jax.numpy as jnp
from jax.experimental import pallas as pl


def kernel(f_atoms, f_bonds, a2a, a2b, a_scope, Wi_w, Wi_b, tune_g, tune_b, tune_p, Wo_w, Wo_b, Wo_g, Wo_bn, Wo_p, Wah_w0, Wah_b0, Wah_w, Wah_bs, Wah_g, Wah_be, Wah_p, Wh0_w, Wh0_b, Wh1_w, Wh1_b, Wh_g, Wh_be, Wh_p):
    raise NotImplementedError("write your pallas kernel here")


# speedup vs baseline: 3.2110x; 3.2110x over previous
import functools

import jax
import jax.numpy as jnp
from jax import lax
from jax.experimental import pallas as pl
from jax.experimental.pallas import tpu as pltpu
from jax.experimental.pallas import tpu_sc as plsc

N = 10000
DEG = 32
A = 133
B = 14
H = 128
D = 3
NB = 320000

NW = 32
APW = 320
NPAD = NW * APW
CA = 128 // DEG
NCHUNK = APW // CA

_SC_MESH = plsc.VectorSubcoreMesh(core_axis_name="c", subcore_axis_name="s")


def _make_gather_sum(width, table_rows, tc_tiling=True):
  ncg = width // 16

  @functools.partial(
      pl.kernel,
      out_type=jax.ShapeDtypeStruct((NPAD, width), jnp.float32),
      mesh=_SC_MESH,
      compiler_params=pltpu.CompilerParams(use_tc_tiling_on_sc=tc_tiling),
      scratch_types=[
          pltpu.VMEM((128,), jnp.int32),
          pltpu.VMEM((128, width), jnp.float32),
          pltpu.VMEM((CA, width), jnp.float32),
          pltpu.SemaphoreType.DMA,
      ],
  )
  def gather_sum(table_hbm, idx_hbm, out_hbm, idx_v, rows_v, outst_v, gsem):
    wid = lax.axis_index("s") * 2 + lax.axis_index("c")
    base = wid * APW

    def body(ch, carry):
      a0 = base + ch * CA
      pltpu.sync_copy(idx_hbm.at[pl.ds(a0 * DEG, 128)], idx_v)
      pltpu.async_copy(table_hbm.at[idx_v], rows_v, gsem).wait()
      for a in range(CA):
        def rbody(r, accs):
          return tuple(
              accs[cg] + rows_v[a * DEG + r, pl.ds(cg * 16, 16)]
              for cg in range(ncg))
        accs = lax.fori_loop(0, DEG, rbody,
                             tuple(jnp.zeros((16,), jnp.float32)
                                   for _ in range(ncg)))
        for cg in range(ncg):
          outst_v[a, pl.ds(cg * 16, 16)] = accs[cg]
      pltpu.sync_copy(outst_v, out_hbm.at[pl.ds(a0, CA)])
      return carry

    lax.fori_loop(0, NCHUNK, body, 0)

  return gather_sum


_gather_sum_msg = _make_gather_sum(H, NPAD)
_gather_sum_bond = _make_gather_sum(16, NB, tc_tiling=False)


def _ln_prelu(x, g, b, p):
  m = x.mean(-1, keepdims=True)
  v = ((x - m) ** 2).mean(-1, keepdims=True)
  y = (x - m) * lax.rsqrt(v + 1e-5) * g + b
  return jnp.where(y >= 0, y, p * y)


BLK = 1280
GRID = NPAD // BLK


def _row_spec(w):
  return pl.BlockSpec((BLK, w), lambda i: (i, 0))


def _w_spec(shape):
  return pl.BlockSpec(shape, lambda i: (0, 0))


def _tc_in_body(fa_ref, w_ref, b_ref, o_ref):
  o_ref[...] = jnp.dot(fa_ref[...], w_ref[...],
                       preferred_element_type=jnp.float32) + b_ref[...]

  @pl.when(pl.program_id(0) == 0)
  def _():
    o_ref[0:1, :] = jnp.zeros((1, H), jnp.float32)


def _tc_in(fa_p, wi_p, wi_b):
  return pl.pallas_call(
      _tc_in_body,
      grid=(GRID,),
      in_specs=[_row_spec(fa_p.shape[1]), _w_spec(wi_p.shape), _w_spec((1, H))],
      out_specs=_row_spec(H),
      out_shape=jax.ShapeDtypeStruct((NPAD, H), jnp.float32),
  )(fa_p, wi_p, wi_b)


def _tc_depth_body(asum_ref, bsum_ref, msg_ref, w0a_ref, w0b_ref, b0_ref,
                   g_ref, be_ref, p_ref, w1_ref, b1_ref, o_ref):
  m = (jnp.dot(asum_ref[...], w0a_ref[...], preferred_element_type=jnp.float32)
       + jnp.dot(bsum_ref[...], w0b_ref[...], preferred_element_type=jnp.float32)
       + b0_ref[...])
  m = _ln_prelu(m, g_ref[...], be_ref[...], p_ref[...])
  m = jnp.dot(m, w1_ref[...], preferred_element_type=jnp.float32) + b1_ref[...]
  o_ref[...] = msg_ref[...] + m

  @pl.when(pl.program_id(0) == 0)
  def _():
    o_ref[0:1, :] = jnp.zeros((1, H), jnp.float32)


def _tc_depth(asum, bsum, msg, w0a, w0b, b0, g, be, p, w1, b1):
  return pl.pallas_call(
      _tc_depth_body,
      grid=(GRID,),
      in_specs=[
          _row_spec(H), _row_spec(16), _row_spec(H),
          _w_spec((H, H)), _w_spec((16, H)), _w_spec((1, H)),
          _w_spec((1, H)), _w_spec((1, H)), _w_spec((1, H)),
          _w_spec((H, H)), _w_spec((1, H)),
      ],
      out_specs=_row_spec(H),
      out_shape=jax.ShapeDtypeStruct((NPAD, H), jnp.float32),
  )(asum, bsum, msg, w0a, w0b, b0, g, be, p, w1, b1)


def _tc_final_body(asum_ref, fa_ref, tg_ref, tb_ref, tp_ref,
                   w0_ref, b0_ref, w1_ref, bs1_ref, w2_ref, bs2_ref,
                   g0_ref, be0_ref, p0_ref, g1_ref, be1_ref, p1_ref,
                   g2_ref, be2_ref, p2_ref,
                   woa_ref, wob_ref, wo_b_ref, wo_g_ref, wo_bn_ref, wo_p_ref,
                   o_ref):
  a_msg = _ln_prelu(asum_ref[...], tg_ref[...], tb_ref[...], tp_ref[...])
  cc = jnp.dot(fa_ref[...], w0_ref[...],
               preferred_element_type=jnp.float32) + b0_ref[...]
  cc = _ln_prelu(cc, g0_ref[...], be0_ref[...], p0_ref[...])
  cc = jnp.dot(cc, w1_ref[...], preferred_element_type=jnp.float32) + bs1_ref[...]
  cc = _ln_prelu(cc, g1_ref[...], be1_ref[...], p1_ref[...])
  cc = jnp.dot(cc, w2_ref[...], preferred_element_type=jnp.float32) + bs2_ref[...]
  cc = _ln_prelu(cc, g2_ref[...], be2_ref[...], p2_ref[...])
  h = (jnp.dot(cc, woa_ref[...], preferred_element_type=jnp.float32)
       + jnp.dot(a_msg, wob_ref[...], preferred_element_type=jnp.float32)
       + wo_b_ref[...])
  o_ref[...] = _ln_prelu(h, wo_g_ref[...], wo_bn_ref[...], wo_p_ref[...])


def _tc_final(asum, fa_p, args):
  ap = fa_p.shape[1]
  vec = _w_spec((1, H))
  return pl.pallas_call(
      _tc_final_body,
      grid=(GRID,),
      in_specs=[
          _row_spec(H), _row_spec(ap),
          vec, vec, vec,
          _w_spec((ap, H)), vec, _w_spec((H, H)), vec, _w_spec((H, H)), vec,
          vec, vec, vec, vec, vec, vec, vec, vec, vec,
          _w_spec((H, H)), _w_spec((H, H)), vec, vec, vec, vec,
      ],
      out_specs=_row_spec(H),
      out_shape=jax.ShapeDtypeStruct((NPAD, H), jnp.float32),
  )(asum, fa_p, *args)


def _rowvec(x):
  return jnp.reshape(x, (1, -1)).astype(jnp.float32)


def _scalar_row(x):
  return jnp.broadcast_to(jnp.reshape(x, (1, 1)), (1, H)).astype(jnp.float32)


def kernel(f_atoms, f_bonds, a2a, a2b, a_scope, Wi_w, Wi_b, tune_g, tune_b,
           tune_p, Wo_w, Wo_b, Wo_g, Wo_bn, Wo_p, Wah_w0, Wah_b0, Wah_w,
           Wah_bs, Wah_g, Wah_be, Wah_p, Wh0_w, Wh0_b, Wh1_w, Wh1_b, Wh_g,
           Wh_be, Wh_p):
  ap = 136
  fa_p = jnp.pad(f_atoms, ((0, NPAD - N), (0, ap - A)))
  fb_p = jnp.pad(f_bonds, ((0, 0), (0, 16 - B)))
  a2a_f = jnp.pad(jnp.reshape(a2a, (-1,)), (0, NPAD * DEG - N * DEG))
  a2b_f = jnp.pad(jnp.reshape(a2b, (-1,)), (0, NPAD * DEG - N * DEG))

  wi_p = jnp.pad(Wi_w, ((0, ap - A), (0, 0)))

  msg = _tc_in(fa_p, wi_p, _rowvec(Wi_b))
  bsum = _gather_sum_bond(fb_p, a2b_f)

  for d in range(D):
    asum = _gather_sum_msg(msg, a2a_f)
    w0a = Wh0_w[d][:H]
    w0b = jnp.pad(Wh0_w[d][H:], ((0, 2), (0, 0)))
    msg = _tc_depth(asum, bsum, msg, w0a, w0b, _rowvec(Wh0_b[d]),
                    _rowvec(Wh_g[d]), _rowvec(Wh_be[d]), _scalar_row(Wh_p[d]),
                    Wh1_w[d], _rowvec(Wh1_b[d]))

  asum = _gather_sum_msg(msg, a2a_f)

  final_args = [
      _rowvec(tune_g), _rowvec(tune_b), _scalar_row(tune_p),
      jnp.pad(Wah_w0, ((0, ap - A), (0, 0))), _rowvec(Wah_b0),
      Wah_w[0], _rowvec(Wah_bs[0]), Wah_w[1], _rowvec(Wah_bs[1]),
      _rowvec(Wah_g[0]), _rowvec(Wah_be[0]), _scalar_row(Wah_p[0]),
      _rowvec(Wah_g[1]), _rowvec(Wah_be[1]), _scalar_row(Wah_p[1]),
      _rowvec(Wah_g[2]), _rowvec(Wah_be[2]), _scalar_row(Wah_p[2]),
      Wo_w[:H], Wo_w[H:], _rowvec(Wo_b), _rowvec(Wo_g), _rowvec(Wo_bn),
      _scalar_row(Wo_p),
  ]
  h = _tc_final(asum, fa_p, final_args)
  return (h[:N], a_scope)


# speedup vs baseline: 3.7814x; 1.1776x over previous
import functools

import jax
import jax.numpy as jnp
from jax import lax
from jax.experimental import pallas as pl
from jax.experimental.pallas import tpu as pltpu
from jax.experimental.pallas import tpu_sc as plsc

N = 10000
DEG = 32
A = 133
B = 14
H = 128
D = 3
NB = 320000

NW = 32
APW = 320
NPAD = NW * APW
CA = 128 // DEG
NCHUNK = APW // CA

_SC_MESH = plsc.VectorSubcoreMesh(core_axis_name="c", subcore_axis_name="s")


def _make_gather_sum(width, table_rows, tc_tiling=True):
  ncg = width // 16

  @functools.partial(
      pl.kernel,
      out_type=jax.ShapeDtypeStruct((NPAD, width), jnp.float32),
      mesh=_SC_MESH,
      compiler_params=pltpu.CompilerParams(use_tc_tiling_on_sc=tc_tiling),
      scratch_types=[
          pltpu.VMEM((NCHUNK, 128), jnp.int32),
          pltpu.VMEM((2, 128, width), jnp.float32),
          pltpu.VMEM((APW, width), jnp.float32),
          pltpu.SemaphoreType.DMA,
          pltpu.SemaphoreType.DMA,
          pltpu.SemaphoreType.DMA,
      ],
  )
  def gather_sum(table_hbm, idx_hbm, out_hbm, idx_v, rows_v, out_v,
                 gsem0, gsem1, osem):
    wid = lax.axis_index("s") * 2 + lax.axis_index("c")
    base = wid * APW
    pltpu.sync_copy(idx_hbm.at[pl.ds(wid * NCHUNK, NCHUNK)], idx_v)
    gsems = (gsem0, gsem1)
    pltpu.async_copy(table_hbm.at[idx_v.at[0]], rows_v.at[0], gsem0)

    def accum(ch, buf):
      def abody(a, _):
        def rbody(r, accs):
          return tuple(
              accs[cg] + rows_v[buf, a * DEG + r, pl.ds(cg * 16, 16)]
              for cg in range(ncg))
        accs = lax.fori_loop(0, DEG, rbody,
                             tuple(jnp.zeros((16,), jnp.float32)
                                   for _ in range(ncg)),
                             unroll=True)
        row = ch * CA + a
        for cg in range(ncg):
          out_v[row, pl.ds(cg * 16, 16)] = accs[cg]
        return 0

      lax.fori_loop(0, CA, abody, 0)

    def body(ch2, carry):
      for b in range(2):
        ch = ch2 * 2 + b
        nxt = ch + 1

        @pl.when(nxt < NCHUNK)
        def _():
          pltpu.async_copy(table_hbm.at[idx_v.at[nxt]],
                           rows_v.at[(b + 1) % 2], gsems[(b + 1) % 2])
        pltpu.make_async_copy(table_hbm.at[idx_v.at[ch]],
                              rows_v.at[b], gsems[b]).wait()
        accum(ch, b)
      return carry

    lax.fori_loop(0, NCHUNK // 2, body, 0)
    pltpu.async_copy(out_v, out_hbm.at[pl.ds(base, APW)], osem).wait()

  return gather_sum


_gather_sum_msg = _make_gather_sum(H, NPAD)
_gather_sum_bond = _make_gather_sum(16, NB, tc_tiling=False)


def _ln_prelu(x, g, b, p):
  m = x.mean(-1, keepdims=True)
  v = ((x - m) ** 2).mean(-1, keepdims=True)
  y = (x - m) * lax.rsqrt(v + 1e-5) * g + b
  return jnp.where(y >= 0, y, p * y)


BLK = 1280
GRID = NPAD // BLK


def _row_spec(w):
  return pl.BlockSpec((BLK, w), lambda i: (i, 0))


def _w_spec(shape):
  return pl.BlockSpec(shape, lambda i: (0, 0))


def _tc_in_body(fa_ref, w_ref, b_ref, o_ref):
  o_ref[...] = jnp.dot(fa_ref[...], w_ref[...],
                       preferred_element_type=jnp.float32) + b_ref[...]

  @pl.when(pl.program_id(0) == 0)
  def _():
    o_ref[0:1, :] = jnp.zeros((1, H), jnp.float32)


def _tc_in(fa_p, wi_p, wi_b):
  return pl.pallas_call(
      _tc_in_body,
      grid=(GRID,),
      in_specs=[_row_spec(fa_p.shape[1]), _w_spec(wi_p.shape), _w_spec((1, H))],
      out_specs=_row_spec(H),
      out_shape=jax.ShapeDtypeStruct((NPAD, H), jnp.float32),
  )(fa_p, wi_p, wi_b)


def _tc_depth_body(asum_ref, bsum_ref, msg_ref, w0a_ref, w0b_ref, b0_ref,
                   g_ref, be_ref, p_ref, w1_ref, b1_ref, o_ref):
  m = (jnp.dot(asum_ref[...], w0a_ref[...], preferred_element_type=jnp.float32)
       + jnp.dot(bsum_ref[...], w0b_ref[...], preferred_element_type=jnp.float32)
       + b0_ref[...])
  m = _ln_prelu(m, g_ref[...], be_ref[...], p_ref[...])
  m = jnp.dot(m, w1_ref[...], preferred_element_type=jnp.float32) + b1_ref[...]
  o_ref[...] = msg_ref[...] + m

  @pl.when(pl.program_id(0) == 0)
  def _():
    o_ref[0:1, :] = jnp.zeros((1, H), jnp.float32)


def _tc_depth(asum, bsum, msg, w0a, w0b, b0, g, be, p, w1, b1):
  return pl.pallas_call(
      _tc_depth_body,
      grid=(GRID,),
      in_specs=[
          _row_spec(H), _row_spec(16), _row_spec(H),
          _w_spec((H, H)), _w_spec((16, H)), _w_spec((1, H)),
          _w_spec((1, H)), _w_spec((1, H)), _w_spec((1, H)),
          _w_spec((H, H)), _w_spec((1, H)),
      ],
      out_specs=_row_spec(H),
      out_shape=jax.ShapeDtypeStruct((NPAD, H), jnp.float32),
  )(asum, bsum, msg, w0a, w0b, b0, g, be, p, w1, b1)


def _tc_final_body(asum_ref, fa_ref, tg_ref, tb_ref, tp_ref,
                   w0_ref, b0_ref, w1_ref, bs1_ref, w2_ref, bs2_ref,
                   g0_ref, be0_ref, p0_ref, g1_ref, be1_ref, p1_ref,
                   g2_ref, be2_ref, p2_ref,
                   woa_ref, wob_ref, wo_b_ref, wo_g_ref, wo_bn_ref, wo_p_ref,
                   o_ref):
  a_msg = _ln_prelu(asum_ref[...], tg_ref[...], tb_ref[...], tp_ref[...])
  cc = jnp.dot(fa_ref[...], w0_ref[...],
               preferred_element_type=jnp.float32) + b0_ref[...]
  cc = _ln_prelu(cc, g0_ref[...], be0_ref[...], p0_ref[...])
  cc = jnp.dot(cc, w1_ref[...], preferred_element_type=jnp.float32) + bs1_ref[...]
  cc = _ln_prelu(cc, g1_ref[...], be1_ref[...], p1_ref[...])
  cc = jnp.dot(cc, w2_ref[...], preferred_element_type=jnp.float32) + bs2_ref[...]
  cc = _ln_prelu(cc, g2_ref[...], be2_ref[...], p2_ref[...])
  h = (jnp.dot(cc, woa_ref[...], preferred_element_type=jnp.float32)
       + jnp.dot(a_msg, wob_ref[...], preferred_element_type=jnp.float32)
       + wo_b_ref[...])
  o_ref[...] = _ln_prelu(h, wo_g_ref[...], wo_bn_ref[...], wo_p_ref[...])


def _tc_final(asum, fa_p, args):
  ap = fa_p.shape[1]
  vec = _w_spec((1, H))
  return pl.pallas_call(
      _tc_final_body,
      grid=(GRID,),
      in_specs=[
          _row_spec(H), _row_spec(ap),
          vec, vec, vec,
          _w_spec((ap, H)), vec, _w_spec((H, H)), vec, _w_spec((H, H)), vec,
          vec, vec, vec, vec, vec, vec, vec, vec, vec,
          _w_spec((H, H)), _w_spec((H, H)), vec, vec, vec, vec,
      ],
      out_specs=_row_spec(H),
      out_shape=jax.ShapeDtypeStruct((NPAD, H), jnp.float32),
  )(asum, fa_p, *args)


def _rowvec(x):
  return jnp.reshape(x, (1, -1)).astype(jnp.float32)


def _scalar_row(x):
  return jnp.broadcast_to(jnp.reshape(x, (1, 1)), (1, H)).astype(jnp.float32)


def kernel(f_atoms, f_bonds, a2a, a2b, a_scope, Wi_w, Wi_b, tune_g, tune_b,
           tune_p, Wo_w, Wo_b, Wo_g, Wo_bn, Wo_p, Wah_w0, Wah_b0, Wah_w,
           Wah_bs, Wah_g, Wah_be, Wah_p, Wh0_w, Wh0_b, Wh1_w, Wh1_b, Wh_g,
           Wh_be, Wh_p):
  ap = 136
  fa_p = jnp.pad(f_atoms, ((0, NPAD - N), (0, ap - A)))
  fb_p = jnp.pad(f_bonds, ((0, 0), (0, 16 - B)))
  a2a_f = jnp.reshape(
      jnp.pad(jnp.reshape(a2a, (-1,)), (0, NPAD * DEG - N * DEG)), (-1, 128))
  a2b_f = jnp.reshape(
      jnp.pad(jnp.reshape(a2b, (-1,)), (0, NPAD * DEG - N * DEG)), (-1, 128))

  wi_p = jnp.pad(Wi_w, ((0, ap - A), (0, 0)))

  msg = _tc_in(fa_p, wi_p, _rowvec(Wi_b))
  bsum = _gather_sum_bond(fb_p, a2b_f)

  for d in range(D):
    asum = _gather_sum_msg(msg, a2a_f)
    w0a = Wh0_w[d][:H]
    w0b = jnp.pad(Wh0_w[d][H:], ((0, 2), (0, 0)))
    msg = _tc_depth(asum, bsum, msg, w0a, w0b, _rowvec(Wh0_b[d]),
                    _rowvec(Wh_g[d]), _rowvec(Wh_be[d]), _scalar_row(Wh_p[d]),
                    Wh1_w[d], _rowvec(Wh1_b[d]))

  asum = _gather_sum_msg(msg, a2a_f)

  final_args = [
      _rowvec(tune_g), _rowvec(tune_b), _scalar_row(tune_p),
      jnp.pad(Wah_w0, ((0, ap - A), (0, 0))), _rowvec(Wah_b0),
      Wah_w[0], _rowvec(Wah_bs[0]), Wah_w[1], _rowvec(Wah_bs[1]),
      _rowvec(Wah_g[0]), _rowvec(Wah_be[0]), _scalar_row(Wah_p[0]),
      _rowvec(Wah_g[1]), _rowvec(Wah_be[1]), _scalar_row(Wah_p[1]),
      _rowvec(Wah_g[2]), _rowvec(Wah_be[2]), _scalar_row(Wah_p[2]),
      Wo_w[:H], Wo_w[H:], _rowvec(Wo_b), _rowvec(Wo_g), _rowvec(Wo_bn),
      _scalar_row(Wo_p),
  ]
  h = _tc_final(asum, fa_p, final_args)
  return (h[:N], a_scope)
